# baseline (speedup 1.0000x reference)
import functools

import jax
import jax.numpy as jnp
from jax.experimental import pallas as pl
from jax.experimental.pallas import tpu as pltpu
from jax.experimental.pallas import tpu_sc as plsc

_W = 128
_MM_BLOCK = 2048


@functools.partial(jax.jit, static_argnums=(2,))
def _sc_gather(table, idx_2d, num_idx):
    rank = table.shape[1]
    n_steps = idx_2d.shape[0]
    mesh = plsc.VectorSubcoreMesh(core_axis_name="core", subcore_axis_name="subcore")

    @functools.partial(
        pl.kernel,
        out_type=jax.ShapeDtypeStruct((num_idx, rank), table.dtype),
        mesh=mesh,
        compiler_params=pltpu.CompilerParams(use_tc_tiling_on_sc=False),
    )
    def gather_kernel(tbl_hbm, idx_hbm, out_hbm):
        def body(i_vmem, o_vmem):
            pltpu.sync_copy(tbl_hbm.at[i_vmem.at[0]], o_vmem)

        pltpu.emit_pipeline(
            body,
            grid=(n_steps,),
            in_specs=[pl.BlockSpec((1, _W), lambda i: (i, 0))],
            out_specs=[pl.BlockSpec((_W, rank), lambda i: (i, 0))],
            core_axis_name=("core", "subcore"),
            dimension_semantics=(pltpu.PARALLEL,),
        )(idx_hbm, out_hbm)

    return gather_kernel(table, idx_2d)


def _mm_body(a_ref, b_ref, o_ref):
    o_ref[...] = jnp.dot(a_ref[...], b_ref[...],
                         preferred_element_type=jnp.float32)


@jax.jit
def _tc_project(a, b):
    n, k = a.shape
    m = b.shape[1]
    return pl.pallas_call(
        _mm_body,
        grid=(n // _MM_BLOCK,),
        in_specs=[
            pl.BlockSpec((_MM_BLOCK, k), lambda i: (i, 0)),
            pl.BlockSpec((k, m), lambda i: (0, 0)),
        ],
        out_specs=pl.BlockSpec((_MM_BLOCK, m), lambda i: (i, 0)),
        out_shape=jax.ShapeDtypeStruct((n, m), jnp.float32),
    )(a, b)


_NCH = 4


def kernel(x, first_factor, last_factor):
    emb_dim = last_factor.shape[1]
    num_idx = x.size
    idx_2d = x.reshape(-1).astype(jnp.int32).reshape(num_idx // _W, _W)
    rows_per_chunk = idx_2d.shape[0] // _NCH
    outs = []
    for c in range(_NCH):
        idx_c = jax.lax.slice_in_dim(idx_2d, c * rows_per_chunk,
                                     (c + 1) * rows_per_chunk, axis=0)
        g_c = _sc_gather(first_factor, idx_c, rows_per_chunk * _W)
        outs.append(_tc_project(g_c, last_factor))
    out = jnp.concatenate(outs, axis=0)
    return out.reshape(tuple(x.shape) + (emb_dim,))


# speedup vs baseline: 1.3909x; 1.3909x over previous
import functools

import jax
import jax.numpy as jnp
from jax.experimental import pallas as pl
from jax.experimental.pallas import tpu as pltpu
from jax.experimental.pallas import tpu_sc as plsc

_MM_ROWS = 128


@jax.jit
def _sc_gather(table, x):
    rank = table.shape[1]
    n_steps, w = x.shape
    mesh = plsc.VectorSubcoreMesh(core_axis_name="core", subcore_axis_name="subcore")

    @functools.partial(
        pl.kernel,
        out_type=jax.ShapeDtypeStruct((n_steps * w, rank), table.dtype),
        mesh=mesh,
        compiler_params=pltpu.CompilerParams(use_tc_tiling_on_sc=False),
    )
    def gather_kernel(tbl_hbm, idx_hbm, out_hbm):
        def body(i_vmem, o_vmem):
            pltpu.sync_copy(tbl_hbm.at[i_vmem.at[0]], o_vmem)

        pltpu.emit_pipeline(
            body,
            grid=(n_steps,),
            in_specs=[pl.BlockSpec((1, w), lambda i: (i, 0))],
            out_specs=[pl.BlockSpec((w, rank), lambda i: (i, 0))],
            core_axis_name=("core", "subcore"),
            dimension_semantics=(pltpu.PARALLEL,),
        )(idx_hbm, out_hbm)

    return gather_kernel(table, x)


def _mm_body(a_ref, b_ref, o_ref):
    br, w, m = o_ref.shape
    res = jnp.dot(a_ref[...], b_ref[...], preferred_element_type=jnp.float32)
    o_ref[...] = res.reshape(br, w, m)


@functools.partial(jax.jit, static_argnums=(2,))
def _tc_project(a, b, n_rows):
    n, k = a.shape
    m = b.shape[1]
    w = n // n_rows
    return pl.pallas_call(
        _mm_body,
        grid=(n_rows // _MM_ROWS,),
        in_specs=[
            pl.BlockSpec((_MM_ROWS * w, k), lambda i: (i, 0)),
            pl.BlockSpec((k, m), lambda i: (0, 0)),
        ],
        out_specs=pl.BlockSpec((_MM_ROWS, w, m), lambda i: (i, 0, 0)),
        out_shape=jax.ShapeDtypeStruct((n_rows, w, m), jnp.float32),
    )(a, b)


def kernel(x, first_factor, last_factor):
    x_i32 = x.astype(jnp.int32)
    gathered = _sc_gather(first_factor, x_i32)
    out = _tc_project(gathered, last_factor, x.shape[0])
    return out.reshape(tuple(x.shape) + (last_factor.shape[1],))


# speedup vs baseline: 1.5442x; 1.1103x over previous
import functools

import jax
import jax.numpy as jnp
from jax.experimental import pallas as pl
from jax.experimental.pallas import tpu as pltpu
from jax.experimental.pallas import tpu_sc as plsc

_MM_ROWS = 128


@jax.jit
def _sc_gather(table, x):
    rank = table.shape[1]
    n_steps, w = x.shape
    mesh = plsc.VectorSubcoreMesh(core_axis_name="core", subcore_axis_name="subcore")

    @functools.partial(
        pl.kernel,
        out_type=jax.ShapeDtypeStruct((n_steps * w, rank), table.dtype),
        mesh=mesh,
        compiler_params=pltpu.CompilerParams(use_tc_tiling_on_sc=False),
    )
    def gather_kernel(tbl_hbm, idx_hbm, out_hbm):
        def body(i_vmem, o_vmem):
            pltpu.sync_copy(tbl_hbm.at[i_vmem.at[0]], o_vmem)

        pltpu.emit_pipeline(
            body,
            grid=(n_steps,),
            in_specs=[pl.BlockSpec((1, w), lambda i: (i, 0))],
            out_specs=[pl.BlockSpec((w, rank), lambda i: (i, 0))],
            core_axis_name=("core", "subcore"),
            dimension_semantics=(pltpu.PARALLEL,),
        )(idx_hbm, out_hbm)

    return gather_kernel(table, x)


def _mm_body(a_ref, b_ref, o_ref):
    br, w, m = o_ref.shape
    res = jnp.dot(a_ref[...], b_ref[...], preferred_element_type=jnp.float32)
    o_ref[...] = res.reshape(br, w, m)


@functools.partial(jax.jit, static_argnums=(2,))
def _tc_project(a, b, n_rows):
    n, k = a.shape
    m = b.shape[1]
    w = n // n_rows
    return pl.pallas_call(
        _mm_body,
        grid=(n_rows // _MM_ROWS,),
        in_specs=[
            pl.BlockSpec((_MM_ROWS * w, k), lambda i: (i, 0)),
            pl.BlockSpec((k, m), lambda i: (0, 0)),
        ],
        out_specs=pl.BlockSpec((_MM_ROWS, w, m), lambda i: (i, 0, 0)),
        out_shape=jax.ShapeDtypeStruct((n_rows, w, m), jnp.float32),
    )(a, b)


def kernel(x, first_factor, last_factor):
    num_idx = x.size
    idx_2d = x.reshape(-1).astype(jnp.int32).reshape(num_idx // 128, 128)
    gathered = _sc_gather(first_factor, idx_2d)
    return _tc_project(gathered, last_factor, x.shape[0])
